# Initial kernel scaffold; baseline (speedup 1.0000x reference)
import jax
import jax.numpy as jnp
from jax.experimental import pallas as pl


def kernel(x, W, b, down_value, indices, down_row, down_col):
    raise NotImplementedError("write your pallas kernel here")


# speedup vs baseline: 3.3732x; 3.3732x over previous
import functools

import jax
import jax.numpy as jnp
from jax import lax
from jax.experimental import pallas as pl
from jax.experimental.pallas import tpu as pltpu
from jax.experimental.pallas import tpu_sc as plsc

BATCH = 4
N_NODES = 50000
M_NODES = 12500
SEQ = 9
CH = 128
NNZ = 37500

BN = 400


def _mm_body(x_ref, w_ref, y_ref):
    xb = x_ref[...].reshape(BATCH * BN, CH)
    y = jnp.dot(xb, w_ref[...], preferred_element_type=jnp.float32)
    for s in range(SEQ):
        for b in range(BATCH):
            y_ref[s * BATCH + b] = y[b * BN:(b + 1) * BN, s * CH:(s + 1) * CH]


def _stage1(x, w_all):
    return pl.pallas_call(
        _mm_body,
        grid=(N_NODES // BN,),
        in_specs=[
            pl.BlockSpec((BATCH, BN, CH), lambda i: (0, i, 0)),
            pl.BlockSpec((CH, SEQ * CH), lambda i: (0, 0)),
        ],
        out_specs=pl.BlockSpec((SEQ * BATCH, BN, CH), lambda i: (0, i, 0)),
        out_shape=jax.ShapeDtypeStruct((SEQ * BATCH, N_NODES, CH), jnp.float32),
    )(x, w_all)


CN = 80
NCHUNK = N_NODES // CN
NWORK = 32

@functools.lru_cache(maxsize=None)
def _mesh():
    return plsc.VectorSubcoreMesh(core_axis_name="c", subcore_axis_name="s")


@functools.lru_cache(maxsize=None)
def _stage2_kernel():
    return pl.kernel(
        _stage2_body,
        out_type=jax.ShapeDtypeStruct((BATCH, N_NODES, CH), jnp.float32),
        mesh=_mesh(),
        scratch_types=[
            pltpu.VMEM((SEQ, CN), jnp.int32),
            pltpu.VMEM((BATCH, CN, CH), jnp.float32),
            pltpu.VMEM((BATCH, CN, CH), jnp.float32),
            pltpu.VMEM((CH,), jnp.float32),
            pltpu.SemaphoreType.DMA,
        ],
    )


def _stage2_body(y_hbm, idxt_hbm, bias_hbm, o_hbm, idx_v, acc_v, buf_v, bias_v, sem):
    wid = lax.axis_index("s") * 2 + lax.axis_index("c")
    pltpu.sync_copy(bias_hbm, bias_v)

    def chunk_body(i, carry):
        ck = wid + i * NWORK

        @pl.when(ck < NCHUNK)
        def _():
            base = ck * CN
            for s in range(SEQ):
                pltpu.sync_copy(
                    idxt_hbm.at[pl.ds(s * N_NODES + base, CN)], idx_v.at[s])
            descs = [
                pltpu.async_copy(y_hbm.at[b].at[idx_v.at[0]], acc_v.at[b], sem)
                for b in range(BATCH)
            ]
            for d in descs:
                d.wait()
            for s in range(1, SEQ):
                descs = [
                    pltpu.async_copy(
                        y_hbm.at[s * BATCH + b].at[idx_v.at[s]], buf_v.at[b], sem)
                    for b in range(BATCH)
                ]
                for d in descs:
                    d.wait()

                def add_body(r, c):
                    for b in range(BATCH):
                        for j in range(CH // 16):
                            sl = (b, r, pl.ds(j * 16, 16))
                            acc_v[sl] = acc_v[sl] + buf_v[sl]
                    return c

                lax.fori_loop(0, CN, add_body, 0)

            def act_body(r, c):
                for b in range(BATCH):
                    for j in range(CH // 16):
                        sl = (b, r, pl.ds(j * 16, 16))
                        v = acc_v[sl] + bias_v[pl.ds(j * 16, 16)]
                        acc_v[sl] = jnp.where(v > 0, v, jnp.exp(v) - 1.0)
                return c

            lax.fori_loop(0, CN, act_body, 0)
            for b in range(BATCH):
                pltpu.sync_copy(acc_v.at[b], o_hbm.at[b, pl.ds(base, CN)])

        return carry

    lax.fori_loop(0, (NCHUNK + NWORK - 1) // NWORK, chunk_body, 0)


EC = 128
NEC = (NNZ + EC - 1) // EC
EP = NEC * EC
M_PAD = 12544
RZ = 128
NRC = M_PAD // RZ


@functools.lru_cache(maxsize=None)
def _stage3_kernel():
    return pl.kernel(
        _stage3_body,
        out_type=jax.ShapeDtypeStruct((BATCH * M_PAD, CH), jnp.float32),
        mesh=_mesh(),
        scratch_types=[
            pltpu.VMEM((1, EC), jnp.int32),
            pltpu.VMEM((EC,), jnp.int32),
            pltpu.VMEM((EC * 16,), jnp.float32),
            pltpu.VMEM((EC, CH), jnp.float32),
            pltpu.VMEM_SHARED((M_PAD, CH), jnp.float32),
            pltpu.SemaphoreType.DMA,
        ],
    )


def _stage3_body(o_hbm, colp_hbm, rowp_hbm, valp_hbm, p_hbm,
                 row_v, col_v, val_v, g_v, shared, sem):
    cid = lax.axis_index("c")
    sid = lax.axis_index("s")

    for bp in range(2):
        b = bp * 2 + cid

        def zb(r, c):
            for j in range(CH // 16):
                g_v[r, pl.ds(j * 16, 16)] = jnp.zeros((16,), jnp.float32)
            return c

        lax.fori_loop(0, RZ, zb, 0)

        def zchunk(i, c):
            ck = sid + i * 16

            @pl.when(ck < NRC)
            def _():
                pltpu.sync_copy(g_v, shared.at[pl.ds(ck * RZ, RZ)])

            return c

        lax.fori_loop(0, (NRC + 15) // 16, zchunk, 0)
        plsc.subcore_barrier()

        def echunk(i, c):
            ck = sid + i * 16

            @pl.when(ck < NEC)
            def _():
                e0 = ck * EC
                pltpu.sync_copy(colp_hbm.at[pl.ds(e0, EC)], col_v)
                pltpu.sync_copy(rowp_hbm.at[pl.ds(e0, EC)], row_v.at[0])
                pltpu.sync_copy(valp_hbm.at[pl.ds(e0 * 16, EC * 16)], val_v)
                off = b * N_NODES
                for j in range(EC // 16):
                    sl = pl.ds(j * 16, 16)
                    col_v[sl] = col_v[sl] + off
                pltpu.async_copy(o_hbm.at[col_v], g_v, sem).wait()

                def scale(e, cc):
                    vk = val_v[pl.ds(e * 16, 16)]
                    for j in range(CH // 16):
                        sl = (e, pl.ds(j * 16, 16))
                        g_v[sl] = g_v[sl] * vk
                    return cc

                lax.fori_loop(0, EC, scale, 0)
                pltpu.sync_copy(g_v, shared.at[row_v.at[0]], add=True)

            return c

        lax.fori_loop(0, (NEC + 15) // 16, echunk, 0)
        plsc.subcore_barrier()

        def wchunk(i, c):
            ck = sid + i * 16

            @pl.when(ck < NRC)
            def _():
                r0 = ck * RZ
                pltpu.sync_copy(shared.at[pl.ds(r0, RZ)],
                                p_hbm.at[pl.ds(b * M_PAD + r0, RZ)])

            return c

        lax.fori_loop(0, (NRC + 15) // 16, wchunk, 0)
        plsc.subcore_barrier()


def kernel(x, W, b, down_value, indices, down_row, down_col):
    w_all = W.reshape(SEQ, CH, CH).transpose(1, 0, 2).reshape(CH, SEQ * CH)
    y = _stage1(x, w_all)

    idxt = indices.astype(jnp.int32).T.reshape(-1)
    o = _stage2_kernel()(y, idxt, b)

    pad = EP - NNZ
    colp = jnp.pad(down_col.astype(jnp.int32), (0, pad))
    rowp = jnp.pad(down_row.astype(jnp.int32), (0, pad))
    valp = jnp.repeat(jnp.pad(down_value, (0, pad)), 16)
    o2 = o.reshape(BATCH * N_NODES, CH)
    p = _stage3_kernel()(o2, colp, rowp, valp)
    return p.reshape(BATCH, M_PAD, CH)[:, :M_NODES]


# speedup vs baseline: 6.5934x; 1.9546x over previous
import functools

import jax
import jax.numpy as jnp
import numpy as np
from jax import lax
from jax.experimental import pallas as pl
from jax.experimental.pallas import tpu as pltpu
from jax.experimental.pallas import tpu_sc as plsc

BATCH = 4
N_NODES = 50000
M_NODES = 12500
SEQ = 9
CH = 128
NNZ = 37500

BN = 400


def _mm_body(x_ref, w_ref, y_ref):
    xb = x_ref[...].reshape(BATCH * BN, CH)
    y = jnp.dot(xb, w_ref[...], preferred_element_type=jnp.float32)
    yb = y.astype(jnp.bfloat16)
    for s in range(SEQ):
        for p in range(BATCH // 2):
            lo = yb[(2 * p) * BN:(2 * p + 1) * BN, s * CH:(s + 1) * CH]
            hi = yb[(2 * p + 1) * BN:(2 * p + 2) * BN, s * CH:(s + 1) * CH]
            lo32 = jax.lax.bitcast_convert_type(
                lo, jnp.uint16).astype(jnp.int32)
            hi32 = jax.lax.bitcast_convert_type(
                hi, jnp.uint16).astype(jnp.int32)
            y_ref[s * 2 + p] = jnp.bitwise_or(
                lo32, jnp.left_shift(hi32, 16))


def _stage1(x_bf, w_all):
    return pl.pallas_call(
        _mm_body,
        grid=(N_NODES // BN,),
        in_specs=[
            pl.BlockSpec((BATCH, BN, CH), lambda i: (0, i, 0)),
            pl.BlockSpec((CH, SEQ * CH), lambda i: (0, 0)),
        ],
        out_specs=pl.BlockSpec((SEQ * 2, BN, CH), lambda i: (0, i, 0)),
        out_shape=jax.ShapeDtypeStruct((SEQ * 2, N_NODES, CH), jnp.int32),
    )(x_bf, w_all)


CN = 40
NCHUNK = N_NODES // CN
NWORK = 32

@functools.lru_cache(maxsize=None)
def _mesh():
    return plsc.VectorSubcoreMesh(core_axis_name="c", subcore_axis_name="s")


@functools.lru_cache(maxsize=None)
def _stage2_kernel():
    return pl.kernel(
        _stage2_body,
        out_type=jax.ShapeDtypeStruct((BATCH, N_NODES, CH), jnp.float32),
        mesh=_mesh(),
        compiler_params=pltpu.CompilerParams(needs_layout_passes=False),
        scratch_types=[
            pltpu.VMEM((SEQ * CN,), jnp.int32),
            pltpu.VMEM((SEQ * 2, CN, CH), jnp.int32),
            pltpu.VMEM((BATCH, CN, CH), jnp.float32),
            pltpu.VMEM((CH,), jnp.float32),
            pltpu.SemaphoreType.DMA,
        ],
    )


def _bf16_pair_to_f32(tot):
    return plsc.unpack(tot, format=plsc.PackFormat.INTERLEAVED)


def _elu(v):
    return jnp.where(v > 0, v, jnp.exp(v) - 1.0)


def _stage2_body(y_hbm, idxc_hbm, bias_hbm, o_hbm, idx_v, buf_v, res_v,
                 bias_v, sem):
    wid = lax.axis_index("s") * 2 + lax.axis_index("c")
    pltpu.sync_copy(bias_hbm, bias_v)

    def chunk_body(i, carry):
        ck = wid + i * NWORK

        @pl.when(ck < NCHUNK)
        def _():
            base = ck * CN
            pltpu.sync_copy(
                idxc_hbm.at[pl.ds(ck * (SEQ * CN), SEQ * CN)], idx_v)
            descs = []
            for s in range(SEQ):
                for p in range(2):
                    k = s * 2 + p
                    descs.append(pltpu.async_copy(
                        y_hbm.at[k].at[idx_v.at[pl.ds(s * CN, CN)]],
                        buf_v.at[k], sem))
            for d in descs:
                d.wait()

            def row_body(r, c):
                for p in range(2):
                    for g in range(CH // 16):
                        sl16 = pl.ds(g * 16, 16)
                        terms = [
                            plsc.bitcast(
                                buf_v[s * 2 + p, r, sl16], jnp.bfloat16)
                            for s in range(SEQ)
                        ]
                        while len(terms) > 1:
                            nxt = []
                            for t in range(0, len(terms) - 1, 2):
                                nxt.append(terms[t] + terms[t + 1])
                            if len(terms) % 2:
                                nxt.append(terms[-1])
                            terms = nxt
                        lo, hi = _bf16_pair_to_f32(terms[0])
                        bias = bias_v[sl16]
                        res_v[2 * p, r, sl16] = _elu(lo + bias)
                        res_v[2 * p + 1, r, sl16] = _elu(hi + bias)
                return c

            lax.fori_loop(0, CN, row_body, 0)
            for b in range(BATCH):
                pltpu.sync_copy(res_v.at[b], o_hbm.at[b, pl.ds(base, CN)])

        return carry

    lax.fori_loop(0, (NCHUNK + NWORK - 1) // NWORK, chunk_body, 0)


EC = 128
NEC = (NNZ + EC - 1) // EC
EP = NEC * EC
M_PAD = 12544
RZ = 128
NRC = M_PAD // RZ


@functools.lru_cache(maxsize=None)
def _stage3_kernel():
    return pl.kernel(
        _stage3_body,
        out_type=jax.ShapeDtypeStruct((BATCH * M_PAD, CH), jnp.float32),
        mesh=_mesh(),
        scratch_types=[
            pltpu.VMEM((1, EC), jnp.int32),
            pltpu.VMEM((EC,), jnp.int32),
            pltpu.VMEM((EC * 16,), jnp.float32),
            pltpu.VMEM((EC, CH), jnp.float32),
            pltpu.VMEM_SHARED((M_PAD, CH), jnp.float32),
            pltpu.SemaphoreType.DMA,
        ],
    )


def _stage3_body(o_hbm, colp_hbm, rowp_hbm, valp_hbm, p_hbm,
                 row_v, col_v, val_v, g_v, shared, sem):
    cid = lax.axis_index("c")
    sid = lax.axis_index("s")

    for bp in range(2):
        b = bp * 2 + cid

        def zb(r, c):
            for j in range(CH // 16):
                g_v[r, pl.ds(j * 16, 16)] = jnp.zeros((16,), jnp.float32)
            return c

        lax.fori_loop(0, RZ, zb, 0)

        def zchunk(i, c):
            ck = sid + i * 16

            @pl.when(ck < NRC)
            def _():
                pltpu.sync_copy(g_v, shared.at[pl.ds(ck * RZ, RZ)])

            return c

        lax.fori_loop(0, (NRC + 15) // 16, zchunk, 0)
        plsc.subcore_barrier()

        def echunk(i, c):
            ck = sid + i * 16

            @pl.when(ck < NEC)
            def _():
                e0 = ck * EC
                pltpu.sync_copy(colp_hbm.at[pl.ds(e0, EC)], col_v)
                pltpu.sync_copy(rowp_hbm.at[pl.ds(e0, EC)], row_v.at[0])
                pltpu.sync_copy(valp_hbm.at[pl.ds(e0 * 16, EC * 16)], val_v)
                off = b * N_NODES
                for j in range(EC // 16):
                    sl = pl.ds(j * 16, 16)
                    col_v[sl] = col_v[sl] + off
                pltpu.async_copy(o_hbm.at[col_v], g_v, sem).wait()

                def scale(e, cc):
                    vk = val_v[pl.ds(e * 16, 16)]
                    for j in range(CH // 16):
                        sl = (e, pl.ds(j * 16, 16))
                        g_v[sl] = g_v[sl] * vk
                    return cc

                lax.fori_loop(0, EC, scale, 0)
                pltpu.sync_copy(g_v, shared.at[row_v.at[0]], add=True)

            return c

        lax.fori_loop(0, (NEC + 15) // 16, echunk, 0)
        plsc.subcore_barrier()

        def wchunk(i, c):
            ck = sid + i * 16

            @pl.when(ck < NRC)
            def _():
                r0 = ck * RZ
                pltpu.sync_copy(shared.at[pl.ds(r0, RZ)],
                                p_hbm.at[pl.ds(b * M_PAD + r0, RZ)])

            return c

        lax.fori_loop(0, (NRC + 15) // 16, wchunk, 0)
        plsc.subcore_barrier()


def kernel(x, W, b, down_value, indices, down_row, down_col):
    w_all = W.reshape(SEQ, CH, CH).transpose(1, 0, 2).reshape(CH, SEQ * CH)
    y = _stage1(x.astype(jnp.bfloat16), w_all.astype(jnp.bfloat16))

    idxc = (indices.astype(jnp.int32).T
            .reshape(SEQ, NCHUNK, CN).transpose(1, 0, 2).reshape(-1))
    o = _stage2_kernel()(y, idxc, b)

    pad = EP - NNZ
    colp = jnp.pad(down_col.astype(jnp.int32), (0, pad))
    rowp = jnp.pad(down_row.astype(jnp.int32), (0, pad))
    valp = jnp.repeat(jnp.pad(down_value, (0, pad)), 16)
    o2 = o.reshape(BATCH * N_NODES, CH)
    p = _stage3_kernel()(o2, colp, rowp, valp)
    return p.reshape(BATCH, M_PAD, CH)[:, :M_NODES]


# speedup vs baseline: 7.2737x; 1.1032x over previous
import functools

import jax
import jax.numpy as jnp
import numpy as np
from jax import lax
from jax.experimental import pallas as pl
from jax.experimental.pallas import tpu as pltpu
from jax.experimental.pallas import tpu_sc as plsc

BATCH = 4
N_NODES = 50000
M_NODES = 12500
SEQ = 9
CH = 128
NNZ = 37500

BN = 400


def _mm_body(x_ref, w_ref, y_ref):
    xb = x_ref[...].reshape(BATCH * BN, CH)
    y = jnp.dot(xb, w_ref[...], preferred_element_type=jnp.float32)
    yb = y.astype(jnp.bfloat16)
    for s in range(SEQ):
        for p in range(BATCH // 2):
            lo = yb[(2 * p) * BN:(2 * p + 1) * BN, s * CH:(s + 1) * CH]
            hi = yb[(2 * p + 1) * BN:(2 * p + 2) * BN, s * CH:(s + 1) * CH]
            lo32 = jax.lax.bitcast_convert_type(
                lo, jnp.uint16).astype(jnp.int32)
            hi32 = jax.lax.bitcast_convert_type(
                hi, jnp.uint16).astype(jnp.int32)
            y_ref[s * 2 + p] = jnp.bitwise_or(
                lo32, jnp.left_shift(hi32, 16))


def _stage1(x_bf, w_all):
    return pl.pallas_call(
        _mm_body,
        grid=(N_NODES // BN,),
        in_specs=[
            pl.BlockSpec((BATCH, BN, CH), lambda i: (0, i, 0)),
            pl.BlockSpec((CH, SEQ * CH), lambda i: (0, 0)),
        ],
        out_specs=pl.BlockSpec((SEQ * 2, BN, CH), lambda i: (0, i, 0)),
        out_shape=jax.ShapeDtypeStruct((SEQ * 2, N_NODES, CH), jnp.int32),
    )(x_bf, w_all)


CN = 16
CNP = CN
NCHUNK = N_NODES // CN
NWORK = 32

@functools.lru_cache(maxsize=None)
def _mesh():
    return plsc.VectorSubcoreMesh(core_axis_name="c", subcore_axis_name="s")


@functools.lru_cache(maxsize=None)
def _stage2_kernel():
    return pl.kernel(
        _stage2_body,
        out_type=jax.ShapeDtypeStruct((BATCH, N_NODES, CH), jnp.float32),
        mesh=_mesh(),
        compiler_params=pltpu.CompilerParams(needs_layout_passes=False),
        scratch_types=[
            pltpu.VMEM((SEQ * CNP,), jnp.int32),
            pltpu.VMEM((SEQ * CNP,), jnp.int32),
            pltpu.VMEM((2, SEQ * 2, CN, CH), jnp.int32),
            pltpu.VMEM((BATCH, CN, CH), jnp.float32),
            pltpu.VMEM((CH,), jnp.float32),
            pltpu.SemaphoreType.DMA,
            pltpu.SemaphoreType.DMA,
        ],
    )


def _bf16_pair_to_f32(tot):
    return plsc.unpack(tot, format=plsc.PackFormat.INTERLEAVED)


def _elu(v):
    return jnp.where(v > 0, v, jnp.exp(v) - 1.0)


def _stage2_body(y_hbm, idxc_hbm, bias_hbm, o_hbm, idx0_v, idx1_v, buf_v,
                 res_v, bias_v, sem0, sem1):
    wid = lax.axis_index("s") * 2 + lax.axis_index("c")
    sems = (sem0, sem1)
    idxs = (idx0_v, idx1_v)
    pltpu.sync_copy(bias_hbm, bias_v)

    def fire(ck, par):
        pltpu.sync_copy(
            idxc_hbm.at[pl.ds(ck * (SEQ * CNP), SEQ * CNP)], idxs[par])
        for s in range(SEQ):
            for p in range(2):
                k = s * 2 + p
                pltpu.async_copy(
                    y_hbm.at[k].at[idxs[par].at[pl.ds(s * CNP, CN)]],
                    buf_v.at[par, k], sems[par])

    def drain(par):
        for s in range(SEQ):
            for p in range(2):
                k = s * 2 + p
                pltpu.make_async_copy(
                    y_hbm.at[k].at[idxs[par].at[pl.ds(s * CNP, CN)]],
                    buf_v.at[par, k], sems[par]).wait()

    def consume(ck, par):
        drain(par)

        def row_body(r, c):
            for p in range(2):
                for g in range(CH // 16):
                    sl16 = pl.ds(g * 16, 16)
                    terms = [
                        plsc.bitcast(
                            buf_v[par, s * 2 + p, r, sl16], jnp.bfloat16)
                        for s in range(SEQ)
                    ]
                    while len(terms) > 1:
                        nxt = []
                        for t in range(0, len(terms) - 1, 2):
                            nxt.append(terms[t] + terms[t + 1])
                        if len(terms) % 2:
                            nxt.append(terms[-1])
                        terms = nxt
                    lo, hi = _bf16_pair_to_f32(terms[0])
                    bias = bias_v[sl16]
                    res_v[2 * p, r, sl16] = _elu(lo + bias)
                    res_v[2 * p + 1, r, sl16] = _elu(hi + bias)
            return c

        lax.fori_loop(0, CN, row_body, 0)
        base = ck * CN
        for b in range(BATCH):
            pltpu.sync_copy(res_v.at[b], o_hbm.at[b, pl.ds(base, CN)])

    fire(wid, 0)

    def pair_body(j, carry):
        for par in range(2):
            ck = wid + (2 * j + par) * NWORK
            nck = wid + (2 * j + par + 1) * NWORK

            @pl.when(nck < NCHUNK)
            def _():
                fire(nck, 1 - par)

            @pl.when(ck < NCHUNK)
            def _():
                consume(ck, par)

        return carry

    niter = (NCHUNK + NWORK - 1) // NWORK
    lax.fori_loop(0, (niter + 1) // 2, pair_body, 0)


EC = 128
NEC = (NNZ + EC - 1) // EC
EP = NEC * EC
M_PAD = 12544
RZ = 128
NRC = M_PAD // RZ


@functools.lru_cache(maxsize=None)
def _stage3_kernel():
    return pl.kernel(
        _stage3_body,
        out_type=jax.ShapeDtypeStruct((BATCH * M_PAD, CH), jnp.float32),
        mesh=_mesh(),
        scratch_types=[
            pltpu.VMEM((1, EC), jnp.int32),
            pltpu.VMEM((EC,), jnp.int32),
            pltpu.VMEM((EC * 16,), jnp.float32),
            pltpu.VMEM((EC, CH), jnp.float32),
            pltpu.VMEM_SHARED((M_PAD, CH), jnp.float32),
            pltpu.SemaphoreType.DMA,
        ],
    )


def _stage3_body(o_hbm, colp_hbm, rowp_hbm, valp_hbm, p_hbm,
                 row_v, col_v, val_v, g_v, shared, sem):
    cid = lax.axis_index("c")
    sid = lax.axis_index("s")

    for bp in range(2):
        b = bp * 2 + cid

        def zb(r, c):
            for j in range(CH // 16):
                g_v[r, pl.ds(j * 16, 16)] = jnp.zeros((16,), jnp.float32)
            return c

        lax.fori_loop(0, RZ, zb, 0)

        def zchunk(i, c):
            ck = sid + i * 16

            @pl.when(ck < NRC)
            def _():
                pltpu.sync_copy(g_v, shared.at[pl.ds(ck * RZ, RZ)])

            return c

        lax.fori_loop(0, (NRC + 15) // 16, zchunk, 0)
        plsc.subcore_barrier()

        def echunk(i, c):
            ck = sid + i * 16

            @pl.when(ck < NEC)
            def _():
                e0 = ck * EC
                pltpu.sync_copy(colp_hbm.at[pl.ds(e0, EC)], col_v)
                pltpu.sync_copy(rowp_hbm.at[pl.ds(e0, EC)], row_v.at[0])
                pltpu.sync_copy(valp_hbm.at[pl.ds(e0 * 16, EC * 16)], val_v)
                off = b * N_NODES
                for j in range(EC // 16):
                    sl = pl.ds(j * 16, 16)
                    col_v[sl] = col_v[sl] + off
                pltpu.async_copy(o_hbm.at[col_v], g_v, sem).wait()

                def scale(e, cc):
                    vk = val_v[pl.ds(e * 16, 16)]
                    for j in range(CH // 16):
                        sl = (e, pl.ds(j * 16, 16))
                        g_v[sl] = g_v[sl] * vk
                    return cc

                lax.fori_loop(0, EC, scale, 0)
                pltpu.sync_copy(g_v, shared.at[row_v.at[0]], add=True)

            return c

        lax.fori_loop(0, (NEC + 15) // 16, echunk, 0)
        plsc.subcore_barrier()

        def wchunk(i, c):
            ck = sid + i * 16

            @pl.when(ck < NRC)
            def _():
                r0 = ck * RZ
                pltpu.sync_copy(shared.at[pl.ds(r0, RZ)],
                                p_hbm.at[pl.ds(b * M_PAD + r0, RZ)])

            return c

        lax.fori_loop(0, (NRC + 15) // 16, wchunk, 0)
        plsc.subcore_barrier()


def kernel(x, W, b, down_value, indices, down_row, down_col):
    w_all = W.reshape(SEQ, CH, CH).transpose(1, 0, 2).reshape(CH, SEQ * CH)
    y = _stage1(x.astype(jnp.bfloat16), w_all.astype(jnp.bfloat16))

    idxc = (indices.astype(jnp.int32).T
            .reshape(SEQ, NCHUNK, CN).transpose(1, 0, 2).reshape(-1))
    o = _stage2_kernel()(y, idxc, b)

    pad = EP - NNZ
    colp = jnp.pad(down_col.astype(jnp.int32), (0, pad))
    rowp = jnp.pad(down_row.astype(jnp.int32), (0, pad))
    valp = jnp.repeat(jnp.pad(down_value, (0, pad)), 16)
    o2 = o.reshape(BATCH * N_NODES, CH)
    p = _stage3_kernel()(o2, colp, rowp, valp)
    return p.reshape(BATCH, M_PAD, CH)[:, :M_NODES]


# speedup vs baseline: 8.2906x; 1.1398x over previous
import functools

import jax
import jax.numpy as jnp
import numpy as np
from jax import lax
from jax.experimental import pallas as pl
from jax.experimental.pallas import tpu as pltpu
from jax.experimental.pallas import tpu_sc as plsc

BATCH = 4
N_NODES = 50000
M_NODES = 12500
SEQ = 9
CH = 128
NNZ = 37500

BN = 400


def _mm_body(x_ref, w_ref, y_ref):
    xb = x_ref[...].reshape(BATCH * BN, CH)
    y = jnp.dot(xb, w_ref[...], preferred_element_type=jnp.float32)
    yb = y.astype(jnp.bfloat16)
    for s in range(SEQ):
        for p in range(BATCH // 2):
            lo = yb[(2 * p) * BN:(2 * p + 1) * BN, s * CH:(s + 1) * CH]
            hi = yb[(2 * p + 1) * BN:(2 * p + 2) * BN, s * CH:(s + 1) * CH]
            lo32 = jax.lax.bitcast_convert_type(
                lo, jnp.uint16).astype(jnp.int32)
            hi32 = jax.lax.bitcast_convert_type(
                hi, jnp.uint16).astype(jnp.int32)
            y_ref[s * 2 + p] = jnp.bitwise_or(
                lo32, jnp.left_shift(hi32, 16))


def _stage1(x_bf, w_all):
    return pl.pallas_call(
        _mm_body,
        grid=(N_NODES // BN,),
        in_specs=[
            pl.BlockSpec((BATCH, BN, CH), lambda i: (0, i, 0)),
            pl.BlockSpec((CH, SEQ * CH), lambda i: (0, 0)),
        ],
        out_specs=pl.BlockSpec((SEQ * 2, BN, CH), lambda i: (0, i, 0)),
        out_shape=jax.ShapeDtypeStruct((SEQ * 2, N_NODES, CH), jnp.int32),
    )(x_bf, w_all)


CN = 16
NCHUNK = N_NODES // CN
NWORK = 32
KPW = (NCHUNK + NWORK - 1) // NWORK
IDXW = SEQ * CN

@functools.lru_cache(maxsize=None)
def _mesh():
    return plsc.VectorSubcoreMesh(core_axis_name="c", subcore_axis_name="s")


@functools.lru_cache(maxsize=None)
def _stage2_kernel():
    return pl.kernel(
        _stage2_body,
        out_type=jax.ShapeDtypeStruct((BATCH, N_NODES, CH), jnp.float32),
        mesh=_mesh(),
        compiler_params=pltpu.CompilerParams(needs_layout_passes=False),
        scratch_types=[
            pltpu.VMEM((KPW * IDXW,), jnp.int32),
            pltpu.VMEM((2, SEQ * 2, CN, CH), jnp.int32),
            pltpu.VMEM((BATCH, CN, CH), jnp.float32),
            pltpu.VMEM((CH,), jnp.float32),
            pltpu.SemaphoreType.DMA,
            pltpu.SemaphoreType.DMA,
        ],
    )


def _bf16_pair_to_f32(tot):
    return plsc.unpack(tot, format=plsc.PackFormat.INTERLEAVED)


def _elu(v):
    return jnp.where(v > 0, v, jnp.exp(v) - 1.0)


def _stage2_body(y_hbm, idxc_hbm, bias_hbm, o_hbm, idx_v, buf_v,
                 res_v, bias_v, sem0, sem1):
    wid = lax.axis_index("s") * 2 + lax.axis_index("c")
    sems = (sem0, sem1)
    pltpu.sync_copy(bias_hbm, bias_v)
    pltpu.sync_copy(idxc_hbm.at[pl.ds(wid * (KPW * IDXW), KPW * IDXW)], idx_v)

    def fire(j, par):
        for s in range(SEQ):
            for p in range(2):
                k = s * 2 + p
                pltpu.async_copy(
                    y_hbm.at[k].at[idx_v.at[pl.ds(j * IDXW + s * CN, CN)]],
                    buf_v.at[par, k], sems[par])

    def drain(j, par):
        for s in range(SEQ):
            for p in range(2):
                k = s * 2 + p
                pltpu.make_async_copy(
                    y_hbm.at[k].at[idx_v.at[pl.ds(j * IDXW + s * CN, CN)]],
                    buf_v.at[par, k], sems[par]).wait()

    def consume(ck, j, par):
        drain(j, par)

        def row_body(r, c):
            for p in range(2):
                for g in range(CH // 16):
                    sl16 = pl.ds(g * 16, 16)
                    terms = [
                        plsc.bitcast(
                            buf_v[par, s * 2 + p, r, sl16], jnp.bfloat16)
                        for s in range(SEQ)
                    ]
                    while len(terms) > 1:
                        nxt = []
                        for t in range(0, len(terms) - 1, 2):
                            nxt.append(terms[t] + terms[t + 1])
                        if len(terms) % 2:
                            nxt.append(terms[-1])
                        terms = nxt
                    lo, hi = _bf16_pair_to_f32(terms[0])
                    bias = bias_v[sl16]
                    res_v[2 * p, r, sl16] = _elu(lo + bias)
                    res_v[2 * p + 1, r, sl16] = _elu(hi + bias)
            return c

        lax.fori_loop(0, CN, row_body, 0)
        base = ck * CN
        for b in range(BATCH):
            pltpu.sync_copy(res_v.at[b], o_hbm.at[b, pl.ds(base, CN)])

    fire(0, 0)

    def pair_body(jj, carry):
        for par in range(2):
            j = 2 * jj + par
            ck = wid + j * NWORK
            nck = ck + NWORK

            @pl.when(nck < NCHUNK)
            def _():
                fire(j + 1, 1 - par)

            @pl.when(ck < NCHUNK)
            def _():
                consume(ck, j, par)

        return carry

    lax.fori_loop(0, (KPW + 1) // 2, pair_body, 0)


EC = 128
NEC = (NNZ + EC - 1) // EC
EP = NEC * EC
MR = 24
M_PAD = 12544
RZ = 128
NRC = M_PAD // RZ


@functools.lru_cache(maxsize=None)
def _stage3_kernel():
    return pl.kernel(
        _stage3_body,
        out_type=jax.ShapeDtypeStruct((BATCH * M_PAD, CH), jnp.float32),
        mesh=_mesh(),
        compiler_params=pltpu.CompilerParams(needs_layout_passes=False),
        scratch_types=[
            pltpu.VMEM((2, MR, CH), jnp.int32),
            pltpu.VMEM((EC, CH), jnp.float32),
            pltpu.VMEM_SHARED((M_PAD, CH), jnp.float32),
            pltpu.SemaphoreType.DMA,
            pltpu.SemaphoreType.DMA,
            pltpu.SemaphoreType.DMA,
        ],
    )


def _stage3_body(o_hbm, emeta_hbm, p_hbm, ebuf_v, g_v, shared,
                 semm0, semm1, sem):
    cid = lax.axis_index("c")
    sid = lax.axis_index("s")
    semms = (semm0, semm1)
    nech = (NEC + 15) // 16

    def meta_fire(i, par):
        ck = sid + i * 16
        pltpu.async_copy(emeta_hbm.at[pl.ds(ck * MR, MR)],
                         ebuf_v.at[par], semms[par])

    def meta_wait(i, par):
        ck = sid + i * 16
        pltpu.make_async_copy(emeta_hbm.at[pl.ds(ck * MR, MR)],
                              ebuf_v.at[par], semms[par]).wait()

    for bp in range(2):
        b = bp * 2 + cid

        def zb(r, c):
            for j in range(CH // 16):
                g_v[r, pl.ds(j * 16, 16)] = jnp.zeros((16,), jnp.float32)
            return c

        lax.fori_loop(0, RZ, zb, 0)

        def zchunk(i, c):
            ck = sid + i * 16

            @pl.when(ck < NRC)
            def _():
                pltpu.sync_copy(g_v, shared.at[pl.ds(ck * RZ, RZ)])

            return c

        lax.fori_loop(0, (NRC + 15) // 16, zchunk, 0)
        plsc.subcore_barrier()

        meta_fire(0, 0)

        def epair(ii, c):
            for par in range(2):
                i = 2 * ii + par
                ck = sid + i * 16

                @pl.when(sid + (i + 1) * 16 < NEC)
                def _():
                    meta_fire(i + 1, 1 - par)

                @pl.when(ck < NEC)
                def _():
                    meta_wait(i, par)
                    off = b * N_NODES
                    for j in range(CH // 16):
                        sl = pl.ds(j * 16, 16)
                        ebuf_v[par, 0, sl] = ebuf_v[par, 0, sl] + off
                    pltpu.async_copy(
                        o_hbm.at[ebuf_v.at[par, 0]], g_v, sem).wait()

                    def scale(r, cc):
                        for cg in range(8):
                            vk = plsc.bitcast(
                                ebuf_v[par, 2 + r, pl.ds(cg * 16, 16)],
                                jnp.float32)
                            e = r * 8 + cg
                            for j in range(CH // 16):
                                sl = (e, pl.ds(j * 16, 16))
                                g_v[sl] = g_v[sl] * vk
                        return cc

                    lax.fori_loop(0, 16, scale, 0)
                    pltpu.sync_copy(g_v, shared.at[ebuf_v.at[par, 1]],
                                    add=True)

            return c

        lax.fori_loop(0, (nech + 1) // 2, epair, 0)
        plsc.subcore_barrier()

        def wchunk(i, c):
            ck = sid + i * 16

            @pl.when(ck < NRC)
            def _():
                r0 = ck * RZ
                pltpu.sync_copy(shared.at[pl.ds(r0, RZ)],
                                p_hbm.at[pl.ds(b * M_PAD + r0, RZ)])

            return c

        lax.fori_loop(0, (NRC + 15) // 16, wchunk, 0)
        plsc.subcore_barrier()


def kernel(x, W, b, down_value, indices, down_row, down_col):
    w_all = W.reshape(SEQ, CH, CH).transpose(1, 0, 2).reshape(CH, SEQ * CH)
    y = _stage1(x.astype(jnp.bfloat16), w_all.astype(jnp.bfloat16))

    idxc = (indices.astype(jnp.int32).T
            .reshape(SEQ, NCHUNK, CN).transpose(1, 0, 2).reshape(NCHUNK, IDXW))
    idxc = jnp.pad(idxc, ((0, KPW * NWORK - NCHUNK), (0, 0)))
    idxc = idxc.reshape(KPW, NWORK, IDXW).transpose(1, 0, 2).reshape(-1)
    o = _stage2_kernel()(y, idxc, b)

    pad = EP - NNZ
    colp = jnp.pad(down_col.astype(jnp.int32), (0, pad)).reshape(NEC, 1, CH)
    rowp = jnp.pad(down_row.astype(jnp.int32), (0, pad)).reshape(NEC, 1, CH)
    val16 = jax.lax.bitcast_convert_type(
        jnp.repeat(jnp.pad(down_value, (0, pad)), 16),
        jnp.int32).reshape(NEC, 16, CH)
    zpad = jnp.zeros((NEC, MR - 18, CH), jnp.int32)
    emeta = jnp.concatenate([colp, rowp, val16, zpad], axis=1)
    o2 = o.reshape(BATCH * N_NODES, CH)
    p = _stage3_kernel()(o2, emeta.reshape(NEC * MR, CH))
    return p.reshape(BATCH, M_PAD, CH)[:, :M_NODES]


# speedup vs baseline: 8.7623x; 1.0569x over previous
import functools

import jax
import jax.numpy as jnp
import numpy as np
from jax import lax
from jax.experimental import pallas as pl
from jax.experimental.pallas import tpu as pltpu
from jax.experimental.pallas import tpu_sc as plsc

BATCH = 4
N_NODES = 50000
M_NODES = 12500
SEQ = 9
CH = 128
NNZ = 37500

BN = 400


def _mm_body(x_ref, w_ref, y_ref):
    xb = x_ref[...].reshape(BATCH * BN, CH)
    y = jnp.dot(xb, w_ref[...], preferred_element_type=jnp.float32)
    yb = y.astype(jnp.bfloat16)
    for s in range(SEQ):
        for p in range(BATCH // 2):
            lo = yb[(2 * p) * BN:(2 * p + 1) * BN, s * CH:(s + 1) * CH]
            hi = yb[(2 * p + 1) * BN:(2 * p + 2) * BN, s * CH:(s + 1) * CH]
            lo32 = jax.lax.bitcast_convert_type(
                lo, jnp.uint16).astype(jnp.int32)
            hi32 = jax.lax.bitcast_convert_type(
                hi, jnp.uint16).astype(jnp.int32)
            y_ref[s * 2 + p] = jnp.bitwise_or(
                lo32, jnp.left_shift(hi32, 16))


def _stage1(x_bf, w_all):
    return pl.pallas_call(
        _mm_body,
        grid=(N_NODES // BN,),
        in_specs=[
            pl.BlockSpec((BATCH, BN, CH), lambda i: (0, i, 0)),
            pl.BlockSpec((CH, SEQ * CH), lambda i: (0, 0)),
        ],
        out_specs=pl.BlockSpec((SEQ * 2, BN, CH), lambda i: (0, i, 0)),
        out_shape=jax.ShapeDtypeStruct((SEQ * 2, N_NODES, CH), jnp.int32),
    )(x_bf, w_all)


CN = 16
NCHUNK = N_NODES // CN
NWORK = 32
KPW = (NCHUNK + NWORK - 1) // NWORK
IDXW = SEQ * CN

@functools.lru_cache(maxsize=None)
def _mesh():
    return plsc.VectorSubcoreMesh(core_axis_name="c", subcore_axis_name="s")


@functools.lru_cache(maxsize=None)
def _stage2_kernel():
    return pl.kernel(
        _stage2_body,
        out_type=jax.ShapeDtypeStruct((BATCH, N_NODES, CH), jnp.float32),
        mesh=_mesh(),
        compiler_params=pltpu.CompilerParams(needs_layout_passes=False),
        scratch_types=[
            pltpu.VMEM((KPW * IDXW,), jnp.int32),
            pltpu.VMEM((2, SEQ * 2, CN, CH), jnp.int32),
            pltpu.VMEM((2, BATCH, CN, CH), jnp.float32),
            pltpu.VMEM((CH,), jnp.float32),
            pltpu.SemaphoreType.DMA,
            pltpu.SemaphoreType.DMA,
            pltpu.SemaphoreType.DMA,
            pltpu.SemaphoreType.DMA,
        ],
    )


def _bf16_pair_to_f32(tot):
    return plsc.unpack(tot, format=plsc.PackFormat.INTERLEAVED)


def _elu(v):
    return jnp.where(v > 0, v, jnp.exp(v) - 1.0)


def _stage2_body(y_hbm, idxc_hbm, bias_hbm, o_hbm, idx_v, buf_v,
                 res_v, bias_v, sem0, sem1, osem0, osem1):
    wid = lax.axis_index("s") * 2 + lax.axis_index("c")
    sems = (sem0, sem1)
    osems = (osem0, osem1)
    pltpu.sync_copy(bias_hbm, bias_v)
    pltpu.sync_copy(idxc_hbm.at[pl.ds(wid * (KPW * IDXW), KPW * IDXW)], idx_v)

    def fire(j, par):
        for s in range(SEQ):
            for p in range(2):
                k = s * 2 + p
                pltpu.async_copy(
                    y_hbm.at[k].at[idx_v.at[pl.ds(j * IDXW + s * CN, CN)]],
                    buf_v.at[par, k], sems[par])

    def drain(j, par):
        for s in range(SEQ):
            for p in range(2):
                k = s * 2 + p
                pltpu.make_async_copy(
                    y_hbm.at[k].at[idx_v.at[pl.ds(j * IDXW + s * CN, CN)]],
                    buf_v.at[par, k], sems[par]).wait()

    def drain_out(par):
        for b in range(BATCH):
            pltpu.make_async_copy(
                res_v.at[par, b], o_hbm.at[b, pl.ds(0, CN)],
                osems[par]).wait()

    def consume(ck, j, par):
        drain(j, par)

        @pl.when(j >= 2)
        def _():
            drain_out(par)

        def row_body(r, c):
            for p in range(2):
                for g in range(CH // 16):
                    sl16 = pl.ds(g * 16, 16)
                    terms = [
                        plsc.bitcast(
                            buf_v[par, s * 2 + p, r, sl16], jnp.bfloat16)
                        for s in range(SEQ)
                    ]
                    while len(terms) > 1:
                        nxt = []
                        for t in range(0, len(terms) - 1, 2):
                            nxt.append(terms[t] + terms[t + 1])
                        if len(terms) % 2:
                            nxt.append(terms[-1])
                        terms = nxt
                    lo, hi = _bf16_pair_to_f32(terms[0])
                    bias = bias_v[sl16]
                    res_v[par, 2 * p, r, sl16] = _elu(lo + bias)
                    res_v[par, 2 * p + 1, r, sl16] = _elu(hi + bias)
            return c

        lax.fori_loop(0, CN, row_body, 0)
        base = ck * CN
        for b in range(BATCH):
            pltpu.async_copy(
                res_v.at[par, b], o_hbm.at[b, pl.ds(base, CN)], osems[par])

    fire(0, 0)

    def pair_body(jj, carry):
        for par in range(2):
            j = 2 * jj + par
            ck = wid + j * NWORK
            nck = ck + NWORK

            @pl.when(nck < NCHUNK)
            def _():
                fire(j + 1, 1 - par)

            @pl.when(ck < NCHUNK)
            def _():
                consume(ck, j, par)

        return carry

    lax.fori_loop(0, (KPW + 1) // 2, pair_body, 0)
    drain_out(0)
    drain_out(1)


EC = 128
NEC = (NNZ + EC - 1) // EC
EP = NEC * EC
MR = 24
M_PAD = 12544
RZ = 128
NRC = M_PAD // RZ


@functools.lru_cache(maxsize=None)
def _stage3_kernel():
    return pl.kernel(
        _stage3_body,
        out_type=jax.ShapeDtypeStruct((BATCH * M_PAD, CH), jnp.float32),
        mesh=_mesh(),
        compiler_params=pltpu.CompilerParams(needs_layout_passes=False),
        scratch_types=[
            pltpu.VMEM((2, MR, CH), jnp.int32),
            pltpu.VMEM((EC, CH), jnp.float32),
            pltpu.VMEM_SHARED((M_PAD, CH), jnp.float32),
            pltpu.SemaphoreType.DMA,
            pltpu.SemaphoreType.DMA,
            pltpu.SemaphoreType.DMA,
        ],
    )


def _stage3_body(o_hbm, emeta_hbm, p_hbm, ebuf_v, g_v, shared,
                 semm0, semm1, sem):
    cid = lax.axis_index("c")
    sid = lax.axis_index("s")
    semms = (semm0, semm1)
    nech = (NEC + 15) // 16

    def meta_fire(i, par):
        ck = sid + i * 16
        pltpu.async_copy(emeta_hbm.at[pl.ds(ck * MR, MR)],
                         ebuf_v.at[par], semms[par])

    def meta_wait(i, par):
        ck = sid + i * 16
        pltpu.make_async_copy(emeta_hbm.at[pl.ds(ck * MR, MR)],
                              ebuf_v.at[par], semms[par]).wait()

    for bp in range(2):
        b = bp * 2 + cid

        def zb(r, c):
            for j in range(CH // 16):
                g_v[r, pl.ds(j * 16, 16)] = jnp.zeros((16,), jnp.float32)
            return c

        lax.fori_loop(0, RZ, zb, 0)

        def zchunk(i, c):
            ck = sid + i * 16

            @pl.when(ck < NRC)
            def _():
                pltpu.sync_copy(g_v, shared.at[pl.ds(ck * RZ, RZ)])

            return c

        lax.fori_loop(0, (NRC + 15) // 16, zchunk, 0)
        plsc.subcore_barrier()

        meta_fire(0, 0)

        def epair(ii, c):
            for par in range(2):
                i = 2 * ii + par
                ck = sid + i * 16

                @pl.when(sid + (i + 1) * 16 < NEC)
                def _():
                    meta_fire(i + 1, 1 - par)

                @pl.when(ck < NEC)
                def _():
                    meta_wait(i, par)
                    off = b * N_NODES
                    for j in range(CH // 16):
                        sl = pl.ds(j * 16, 16)
                        ebuf_v[par, 0, sl] = ebuf_v[par, 0, sl] + off
                    pltpu.async_copy(
                        o_hbm.at[ebuf_v.at[par, 0]], g_v, sem).wait()

                    def scale(r, cc):
                        for cg in range(8):
                            vk = plsc.bitcast(
                                ebuf_v[par, 2 + r, pl.ds(cg * 16, 16)],
                                jnp.float32)
                            e = r * 8 + cg
                            for j in range(CH // 16):
                                sl = (e, pl.ds(j * 16, 16))
                                g_v[sl] = g_v[sl] * vk
                        return cc

                    lax.fori_loop(0, 16, scale, 0)
                    pltpu.sync_copy(g_v, shared.at[ebuf_v.at[par, 1]],
                                    add=True)

            return c

        lax.fori_loop(0, (nech + 1) // 2, epair, 0)
        plsc.subcore_barrier()

        def wchunk(i, c):
            ck = sid + i * 16

            @pl.when(ck < NRC)
            def _():
                r0 = ck * RZ
                pltpu.sync_copy(shared.at[pl.ds(r0, RZ)],
                                p_hbm.at[pl.ds(b * M_PAD + r0, RZ)])

            return c

        lax.fori_loop(0, (NRC + 15) // 16, wchunk, 0)
        plsc.subcore_barrier()


def kernel(x, W, b, down_value, indices, down_row, down_col):
    w_all = W.reshape(SEQ, CH, CH).transpose(1, 0, 2).reshape(CH, SEQ * CH)
    y = _stage1(x.astype(jnp.bfloat16), w_all.astype(jnp.bfloat16))

    idxc = (indices.astype(jnp.int32).T
            .reshape(SEQ, NCHUNK, CN).transpose(1, 0, 2).reshape(NCHUNK, IDXW))
    idxc = jnp.pad(idxc, ((0, KPW * NWORK - NCHUNK), (0, 0)))
    idxc = idxc.reshape(KPW, NWORK, IDXW).transpose(1, 0, 2).reshape(-1)
    o = _stage2_kernel()(y, idxc, b)

    pad = EP - NNZ
    colp = jnp.pad(down_col.astype(jnp.int32), (0, pad)).reshape(NEC, 1, CH)
    rowp = jnp.pad(down_row.astype(jnp.int32), (0, pad)).reshape(NEC, 1, CH)
    val16 = jax.lax.bitcast_convert_type(
        jnp.repeat(jnp.pad(down_value, (0, pad)), 16),
        jnp.int32).reshape(NEC, 16, CH)
    zpad = jnp.zeros((NEC, MR - 18, CH), jnp.int32)
    emeta = jnp.concatenate([colp, rowp, val16, zpad], axis=1)
    o2 = o.reshape(BATCH * N_NODES, CH)
    p = _stage3_kernel()(o2, emeta.reshape(NEC * MR, CH))
    return p.reshape(BATCH, M_PAD, CH)[:, :M_NODES]
